# baseline (speedup 1.0000x reference)
import jax
import jax.numpy as jnp
from jax import lax
from jax.experimental import pallas as pl
from jax.experimental.pallas import tpu as pltpu
from jax.experimental.pallas import tpu_sc as plsc

N_TOKEN = 1000000
D_EMBED = 64
PAD_IDX = N_TOKEN - 1
EMB_SCALE = 8.0
BATCH = 4096
HIST_LEN = 200

NC = 2
NS = 16
NW = NC * NS

N_IDX = BATCH * HIST_LEN
PER_W = N_IDX // NW
CHUNK = 128
NCHUNK = PER_W // CHUNK
LANES = 16
COLS = D_EMBED // LANES


def _body(idx_hbm, table_hbm, out_hbm, idx_v, rows_v, gsem):
    wid = lax.axis_index("s") * NC + lax.axis_index("c")
    base = wid * PER_W

    pltpu.sync_copy(idx_hbm.at[wid], idx_v)

    def chunk_body(g, _):
        pltpu.async_copy(table_hbm.at[idx_v.at[g]], rows_v, gsem).wait()

        def scale_rows(r16, _):
            base_r = r16 * LANES
            tvec = idx_v[g, pl.ds(base_r, LANES)]
            svec = jnp.where(tvec == PAD_IDX, 0.0, EMB_SCALE)
            for j in range(LANES):
                s = svec[j]
                for c in range(COLS):
                    sl = pl.ds(c * LANES, LANES)
                    rows_v[base_r + j, sl] = rows_v[base_r + j, sl] * s
            return 0

        lax.fori_loop(0, CHUNK // LANES, scale_rows, 0)

        pltpu.sync_copy(rows_v, out_hbm.at[pl.ds(base + g * CHUNK, CHUNK)])
        return 0

    lax.fori_loop(0, NCHUNK, chunk_body, 0)


_emb_lookup = pl.kernel(
    _body,
    out_type=jax.ShapeDtypeStruct((N_IDX, D_EMBED), jnp.float32),
    mesh=plsc.VectorSubcoreMesh(
        core_axis_name="c", subcore_axis_name="s", num_cores=NC, num_subcores=NS
    ),
    scratch_types=[
        pltpu.VMEM((NCHUNK, CHUNK), jnp.int32),
        pltpu.VMEM((CHUNK, D_EMBED), jnp.float32),
        pltpu.SemaphoreType.DMA,
    ],
    compiler_params=pltpu.CompilerParams(use_tc_tiling_on_sc=False),
)


def kernel(inp_tokens, emb_table):
    idx = inp_tokens.reshape(NW, NCHUNK, CHUNK)
    out = _emb_lookup(idx, emb_table)
    return out.reshape(BATCH, HIST_LEN, D_EMBED)


# speedup vs baseline: 1.1817x; 1.1817x over previous
import jax
import jax.numpy as jnp
from jax import lax
from jax.experimental import pallas as pl
from jax.experimental.pallas import tpu as pltpu
from jax.experimental.pallas import tpu_sc as plsc

N_TOKEN = 1000000
D_EMBED = 64
PAD_IDX = N_TOKEN - 1
EMB_SCALE = 8.0
BATCH = 4096
HIST_LEN = 200

NC = 2
NS = 16
NW = NC * NS

N_IDX = BATCH * HIST_LEN
PER_W = N_IDX // NW
CHUNK = 128
NCHUNK = PER_W // CHUNK
LANES = 16
COLS = D_EMBED // LANES
NBUF = 4


def _body(idx_hbm, table_hbm, out_hbm,
          idx_v, b0, b1, b2, b3,
          g0, g1, g2, g3, s0, s1, s2, s3):
    bufs = (b0, b1, b2, b3)
    gsems = (g0, g1, g2, g3)
    ssems = (s0, s1, s2, s3)

    wid = lax.axis_index("s") * NC + lax.axis_index("c")
    base = wid * PER_W

    pltpu.sync_copy(idx_hbm.at[wid], idx_v)

    def start_gather(g, b):
        pltpu.async_copy(table_hbm.at[idx_v.at[g]], bufs[b], gsems[b])

    def wait_gather(g, b):
        pltpu.make_async_copy(table_hbm.at[idx_v.at[g]], bufs[b], gsems[b]).wait()

    def start_scatter(g, b):
        pltpu.async_copy(bufs[b], out_hbm.at[pl.ds(base + g * CHUNK, CHUNK)], ssems[b])

    def wait_scatter(g, b):
        pltpu.make_async_copy(
            bufs[b], out_hbm.at[pl.ds(base + g * CHUNK, CHUNK)], ssems[b]
        ).wait()

    def scale(g, b):
        buf = bufs[b]

        def scale_rows(r16, _):
            base_r = r16 * LANES
            tvec = idx_v[g, pl.ds(base_r, LANES)]
            svec = jnp.where(tvec == PAD_IDX, 0.0, EMB_SCALE)
            for j in range(LANES):
                s = svec[j]
                for c in range(COLS):
                    sl = pl.ds(c * LANES, LANES)
                    buf[base_r + j, sl] = buf[base_r + j, sl] * s
            return 0

        lax.fori_loop(0, CHUNK // LANES, scale_rows, 0)

    def slot(g, b, wait_sc, start_g):
        bn = (b + 2) % NBUF
        if start_g:
            if wait_sc:
                wait_scatter(g - 2, bn)
            start_gather(g + 2, bn)
        wait_gather(g, b)
        scale(g, b)
        start_scatter(g, b)

    start_gather(0, 0)
    start_gather(1, 1)
    slot(0, 0, wait_sc=False, start_g=True)
    slot(1, 1, wait_sc=False, start_g=True)
    slot(2, 2, wait_sc=True, start_g=True)
    slot(3, 3, wait_sc=True, start_g=True)

    def super_body(G, _):
        g = G * NBUF
        for b in range(NBUF):
            slot(g + b, b, wait_sc=True, start_g=True)
        return 0

    lax.fori_loop(1, NCHUNK // NBUF - 1, super_body, 0)

    gl = NCHUNK - NBUF
    slot(gl + 0, 0, wait_sc=True, start_g=True)
    slot(gl + 1, 1, wait_sc=True, start_g=True)
    slot(gl + 2, 2, wait_sc=False, start_g=False)
    slot(gl + 3, 3, wait_sc=False, start_g=False)

    for b in range(NBUF):
        wait_scatter(gl + b, b)


_emb_lookup = pl.kernel(
    _body,
    out_type=jax.ShapeDtypeStruct((N_IDX, D_EMBED), jnp.float32),
    mesh=plsc.VectorSubcoreMesh(
        core_axis_name="c", subcore_axis_name="s", num_cores=NC, num_subcores=NS
    ),
    scratch_types=(
        [pltpu.VMEM((NCHUNK, CHUNK), jnp.int32)]
        + [pltpu.VMEM((CHUNK, D_EMBED), jnp.float32) for _ in range(NBUF)]
        + [pltpu.SemaphoreType.DMA for _ in range(2 * NBUF)]
    ),
    compiler_params=pltpu.CompilerParams(use_tc_tiling_on_sc=False),
)


def kernel(inp_tokens, emb_table):
    idx = inp_tokens.reshape(NW, NCHUNK, CHUNK)
    out = _emb_lookup(idx, emb_table)
    return out.reshape(BATCH, HIST_LEN, D_EMBED)


# speedup vs baseline: 1.2645x; 1.0701x over previous
import jax
import jax.numpy as jnp
from jax import lax
from jax.experimental import pallas as pl
from jax.experimental.pallas import tpu as pltpu
from jax.experimental.pallas import tpu_sc as plsc

N_TOKEN = 1000000
D_EMBED = 64
PAD_IDX = N_TOKEN - 1
EMB_SCALE = 8.0
BATCH = 4096
HIST_LEN = 200

NC = 2
NS = 16
NW = NC * NS

N_IDX = BATCH * HIST_LEN
PER_W = N_IDX // NW
CHUNK = 128
NCHUNK = PER_W // CHUNK
LANES = 16
COLS = D_EMBED // LANES
NBUF = 3
LOOK = 2


def _body(idx_hbm, table_hbm, out_hbm, idx_v,
          sl0, sl1, sl2,
          b0, b1, b2,
          o0, o1, o2,
          g0, g1, g2,
          s0, s1, s2):
    slists = (sl0, sl1, sl2)
    bufs = (b0, b1, b2)
    obufs = (o0, o1, o2)
    gsems = (g0, g1, g2)
    ssems = (s0, s1, s2)

    wid = lax.axis_index("s") * NC + lax.axis_index("c")
    base = wid * PER_W

    pltpu.sync_copy(idx_hbm.at[wid], idx_v)

    def start_gather(g, b):
        def grp(G, _):
            o = G * LANES
            slists[b][pl.ds(o, LANES)] = lax.shift_right_logical(
                idx_v[g, pl.ds(o, LANES)], 1)
            return 0

        lax.fori_loop(0, CHUNK // LANES, grp, 0)
        pltpu.async_copy(table_hbm.at[slists[b]], bufs[b], gsems[b])

    def wait_gather(b):
        pltpu.make_async_copy(table_hbm.at[slists[b]], bufs[b], gsems[b]).wait()

    def start_scatter(g, b):
        pltpu.async_copy(obufs[b], out_hbm.at[pl.ds(base + g * CHUNK, CHUNK)],
                         ssems[b])

    def wait_scatter(g, b):
        pltpu.make_async_copy(
            obufs[b], out_hbm.at[pl.ds(base + g * CHUNK, CHUNK)], ssems[b]
        ).wait()

    def select_scale(g, b):
        buf = bufs[b]
        obuf = obufs[b]

        def scale_rows(r16, _):
            base_r = r16 * LANES
            tvec = idx_v[g, pl.ds(base_r, LANES)]
            hvec = (tvec & 1) * D_EMBED
            svec = jnp.where(tvec == PAD_IDX, 0.0, EMB_SCALE)
            for j in range(LANES):
                h = hvec[j]
                s = svec[j]
                for c in range(COLS):
                    obuf[base_r + j, pl.ds(c * LANES, LANES)] = (
                        buf[base_r + j, pl.ds(h + c * LANES, LANES)] * s)
            return 0

        lax.fori_loop(0, CHUNK // LANES, scale_rows, 0)

    def slot(g, b, wait_sc, start_g):
        bn = (b + LOOK) % NBUF
        if start_g:
            if wait_sc:
                wait_scatter(g + LOOK - NBUF, bn)
            start_gather(g + LOOK, bn)
        wait_gather(b)
        select_scale(g, b)
        start_scatter(g, b)

    for g in range(LOOK):
        start_gather(g, g)
    slot(0, 0, wait_sc=False, start_g=True)
    slot(1, 1, wait_sc=True, start_g=True)

    def super_body(G, _):
        g = 2 + G * NBUF
        for k in range(NBUF):
            slot(g + k, (2 + k) % NBUF, wait_sc=True, start_g=True)
        return 0

    lax.fori_loop(0, (NCHUNK - 2) // NBUF - 1, super_body, 0)

    ge = NCHUNK - NBUF
    for k in range(NBUF):
        g = ge + k
        slot(g, g % NBUF, wait_sc=True, start_g=(g + LOOK < NCHUNK))

    for k in range(NBUF):
        g = ge + k
        wait_scatter(g, g % NBUF)


_emb_lookup = pl.kernel(
    _body,
    out_type=jax.ShapeDtypeStruct((N_IDX, D_EMBED), jnp.float32),
    mesh=plsc.VectorSubcoreMesh(
        core_axis_name="c", subcore_axis_name="s", num_cores=NC, num_subcores=NS
    ),
    scratch_types=(
        [pltpu.VMEM((NCHUNK, CHUNK), jnp.int32)]
        + [pltpu.VMEM((CHUNK,), jnp.int32) for _ in range(NBUF)]
        + [pltpu.VMEM((CHUNK, 2 * D_EMBED), jnp.float32) for _ in range(NBUF)]
        + [pltpu.VMEM((CHUNK, D_EMBED), jnp.float32) for _ in range(NBUF)]
        + [pltpu.SemaphoreType.DMA for _ in range(2 * NBUF)]
    ),
    compiler_params=pltpu.CompilerParams(use_tc_tiling_on_sc=True),
)


def kernel(inp_tokens, emb_table):
    idx = inp_tokens.reshape(NW, NCHUNK, CHUNK)
    table_pairs = emb_table.reshape(N_TOKEN // 2, 2 * D_EMBED)
    out = _emb_lookup(idx, table_pairs)
    return out.reshape(BATCH, HIST_LEN, D_EMBED)


# speedup vs baseline: 1.3904x; 1.0996x over previous
import jax
import jax.numpy as jnp
from jax import lax
from jax.experimental import pallas as pl
from jax.experimental.pallas import tpu as pltpu
from jax.experimental.pallas import tpu_sc as plsc

N_TOKEN = 1000000
D_EMBED = 64
PAD_IDX = N_TOKEN - 1
EMB_SCALE = 8.0
BATCH = 4096
HIST_LEN = 200

NC = 2
NS = 16
NW = NC * NS

N_IDX = BATCH * HIST_LEN
PER_W = N_IDX // NW
CHUNK = 64
NCHUNK = PER_W // CHUNK
LANES = 16
COLS = D_EMBED // LANES
NBUF = 5
LOOK = 3


def _body(idx_hbm, table_hbm, out_hbm,
          i0, i1, i2, i3, i4,
          sl0, sl1, sl2, sl3, sl4,
          b0, b1, b2, b3, b4,
          o0, o1, o2, o3, o4,
          is0, is1, is2, is3, is4,
          g0, g1, g2, g3, g4,
          s0, s1, s2, s3, s4):
    ibufs = (i0, i1, i2, i3, i4)
    slists = (sl0, sl1, sl2, sl3, sl4)
    bufs = (b0, b1, b2, b3, b4)
    obufs = (o0, o1, o2, o3, o4)
    isems = (is0, is1, is2, is3, is4)
    gsems = (g0, g1, g2, g3, g4)
    ssems = (s0, s1, s2, s3, s4)

    wid = lax.axis_index("s") * NC + lax.axis_index("c")
    base = wid * PER_W

    def start_idx(g, b):
        pltpu.async_copy(idx_hbm.at[wid, g], ibufs[b], isems[b])

    def wait_idx(g, b):
        pltpu.make_async_copy(idx_hbm.at[wid, g], ibufs[b], isems[b]).wait()

    def start_gather(g, b):
        def grp(G, _):
            o = G * LANES
            slists[b][pl.ds(o, LANES)] = lax.shift_right_logical(
                ibufs[b][pl.ds(o, LANES)], 1)
            return 0

        lax.fori_loop(0, CHUNK // LANES, grp, 0)
        pltpu.async_copy(table_hbm.at[slists[b]], bufs[b], gsems[b])

    def wait_gather(b):
        pltpu.make_async_copy(table_hbm.at[slists[b]], bufs[b], gsems[b]).wait()

    def start_scatter(g, b):
        pltpu.async_copy(obufs[b], out_hbm.at[pl.ds(base + g * CHUNK, CHUNK)],
                         ssems[b])

    def wait_scatter(g, b):
        pltpu.make_async_copy(
            obufs[b], out_hbm.at[pl.ds(base + g * CHUNK, CHUNK)], ssems[b]
        ).wait()

    def select_scale(b):
        buf = bufs[b]
        obuf = obufs[b]
        ibuf = ibufs[b]

        def scale_rows(r16, _):
            base_r = r16 * LANES
            tvec = ibuf[pl.ds(base_r, LANES)]
            hvec = (tvec & 1) * D_EMBED
            svec = jnp.where(tvec == PAD_IDX, 0.0, EMB_SCALE)
            for j in range(LANES):
                h = hvec[j]
                s = svec[j]
                for c in range(COLS):
                    obuf[base_r + j, pl.ds(c * LANES, LANES)] = (
                        buf[base_r + j, pl.ds(h + c * LANES, LANES)] * s)
            return 0

        lax.fori_loop(0, CHUNK // LANES, scale_rows, 0)

    def slot(g, b, wait_sc, start_g, start_i):
        bn = (b + LOOK) % NBUF
        if start_g:
            if wait_sc:
                wait_scatter(g + LOOK - NBUF, bn)
            wait_idx(g + LOOK, bn)
            start_gather(g + LOOK, bn)
        wait_gather(b)
        select_scale(b)
        if start_i:
            start_idx(g + NBUF, b)
        start_scatter(g, b)

    for q in range(NBUF):
        start_idx(q, q)
    for g in range(LOOK):
        wait_idx(g, g)
        start_gather(g, g)
    for g in range(LOOK):
        slot(g, g, wait_sc=(g + LOOK >= NBUF), start_g=True, start_i=True)

    K = (NCHUNK - LOOK) // NBUF - 2

    def super_body(G, _):
        g = LOOK + G * NBUF
        for k in range(NBUF):
            slot(g + k, (LOOK + k) % NBUF, wait_sc=True, start_g=True,
                 start_i=True)
        return 0

    lax.fori_loop(0, K, super_body, 0)

    R = NCHUNK - (LOOK + NBUF * K)
    for j in range(R):
        g = NCHUNK - R + j
        slot(g, g % NBUF, wait_sc=True, start_g=(g + LOOK < NCHUNK),
             start_i=(g + NBUF < NCHUNK))

    for j in range(NBUF):
        g = NCHUNK - NBUF + j
        wait_scatter(g, g % NBUF)


_emb_lookup = pl.kernel(
    _body,
    out_type=jax.ShapeDtypeStruct((N_IDX, D_EMBED), jnp.float32),
    mesh=plsc.VectorSubcoreMesh(
        core_axis_name="c", subcore_axis_name="s", num_cores=NC, num_subcores=NS
    ),
    scratch_types=(
        [pltpu.VMEM((CHUNK,), jnp.int32) for _ in range(NBUF)]
        + [pltpu.VMEM((CHUNK,), jnp.int32) for _ in range(NBUF)]
        + [pltpu.VMEM((CHUNK, 2 * D_EMBED), jnp.float32) for _ in range(NBUF)]
        + [pltpu.VMEM((CHUNK, D_EMBED), jnp.float32) for _ in range(NBUF)]
        + [pltpu.SemaphoreType.DMA for _ in range(3 * NBUF)]
    ),
    compiler_params=pltpu.CompilerParams(use_tc_tiling_on_sc=True),
)


def kernel(inp_tokens, emb_table):
    idx = inp_tokens.reshape(NW, NCHUNK, CHUNK)
    table_pairs = emb_table.reshape(N_TOKEN // 2, 2 * D_EMBED)
    out = _emb_lookup(idx, table_pairs)
    return out.reshape(BATCH, HIST_LEN, D_EMBED)
